# Initial kernel scaffold; baseline (speedup 1.0000x reference)
import jax
import jax.numpy as jnp
from jax.experimental import pallas as pl


def kernel(inputs, weight):
    raise NotImplementedError("write your pallas kernel here")


# speedup vs baseline: 4.1425x; 4.1425x over previous
import jax
import jax.numpy as jnp
from jax.experimental import pallas as pl
from jax.experimental.pallas import tpu as pltpu

_K = 8192
_D = 32
_R = 256
_N = 16384
_COMMIT = 0.25


def _vq_body(xsq_ref, wsq_ref, x16_ref, x_ref, w_ref,
             idx_ref, q_ref, part_ref):
    xb = x_ref[...]
    m_t = jax.lax.dot_general(
        w_ref[...], x16_ref[...], (((1,), (1,)), ((), ())),
        preferred_element_type=jnp.float32)
    dist_t = (xsq_ref[...] + wsq_ref[...]) - 2.0 * m_t
    idx = jnp.argmin(dist_t, axis=0)
    idx_ref[...] = idx[:, None]
    iota = jax.lax.broadcasted_iota(jnp.int32, (_R, _K), 1)
    onehot = (iota == idx[:, None]).astype(jnp.float32)
    q = jax.lax.dot_general(
        onehot, w_ref[...], (((1,), (0,)), ((), ())),
        precision=jax.lax.Precision.HIGHEST,
        preferred_element_type=jnp.float32)
    q_ref[...] = xb + (q - xb)
    part_ref[pl.program_id(0), 0] = jnp.sum((q - xb) ** 2)


def kernel(inputs, weight):
    shape = inputs.shape
    flat = inputs.reshape(_N, _D)
    x16 = flat.astype(jnp.bfloat16)
    xsq = jnp.sum(flat ** 2, axis=1)[None, :]
    wsq = jnp.sum(weight ** 2, axis=1)[:, None]
    nblk = _N // _R

    idx, q_st, parts = pl.pallas_call(
        _vq_body,
        grid=(nblk,),
        in_specs=[
            pl.BlockSpec((1, _R), lambda i: (0, i)),
            pl.BlockSpec((_K, 1), lambda i: (0, 0)),
            pl.BlockSpec((_R, _D), lambda i: (i, 0)),
            pl.BlockSpec((_R, _D), lambda i: (i, 0)),
            pl.BlockSpec((_K, _D), lambda i: (0, 0)),
        ],
        out_specs=[
            pl.BlockSpec((_R, 1), lambda i: (i, 0)),
            pl.BlockSpec((_R, _D), lambda i: (i, 0)),
            pl.BlockSpec((nblk, 1), lambda i: (0, 0),
                         memory_space=pltpu.MemorySpace.SMEM),
        ],
        out_shape=[
            jax.ShapeDtypeStruct((_N, 1), jnp.int32),
            jax.ShapeDtypeStruct((_N, _D), jnp.float32),
            jax.ShapeDtypeStruct((nblk, 1), jnp.float32),
        ],
    )(xsq, wsq, x16, flat, weight)

    mse = jnp.sum(parts) / (_N * _D)
    loss_vq = mse + _COMMIT * mse
    return (loss_vq, q_st.reshape(shape), idx.reshape(shape[0], shape[1]))
